# baseline (device time: 29710 ns/iter reference)
import jax
import jax.numpy as jnp
from jax import lax
from jax.experimental import pallas as pl
from jax.experimental.pallas import tpu as pltpu

M = 512
D = 512
K = 2048
MS = 64
MC = 128
NZ = 4


def kernel(dy, W):
    def body(
        dy_ref, w_ref, out_ref,
        dyslab_ref, slab_ref, ycomm_ref,
        copy_sem, y_send, y_recv,
        z_send, z_recv, x_send, x_recv,
    ):
        my_x = lax.axis_index("x")
        my_y = lax.axis_index("y")
        my_z = lax.axis_index("z")
        row0 = MC * my_z + MS * my_x

        dy_copy = pltpu.make_async_copy(
            dy_ref.at[pl.ds(row0, MS), :], dyslab_ref, copy_sem
        )
        dy_copy.start()

        barrier_sem = pltpu.get_barrier_semaphore()
        for dist in (1, 2):
            pl.semaphore_signal(
                barrier_sem, inc=1,
                device_id=(my_x, my_y ^ dist, my_z),
                device_id_type=pl.DeviceIdType.MESH,
            )
        pl.semaphore_signal(
            barrier_sem, inc=1,
            device_id=(1 - my_x, my_y, my_z),
            device_id_type=pl.DeviceIdType.MESH,
        )
        for d in (1, 2, 3):
            pl.semaphore_signal(
                barrier_sem, inc=1,
                device_id=(my_x, my_y, (my_z + d) % NZ),
                device_id_type=pl.DeviceIdType.MESH,
            )
        pl.semaphore_wait(barrier_sem, 6)
        dy_copy.wait()

        slab_ref[:, :] = lax.dot_general(
            dyslab_ref[:, :], w_ref[:, :],
            (((1,), (1,)), ((), ())),
            preferred_element_type=jnp.float32,
        )

        for s, dist in enumerate((1, 2)):
            rdma = pltpu.make_async_remote_copy(
                src_ref=slab_ref,
                dst_ref=ycomm_ref.at[s],
                send_sem=y_send.at[s],
                recv_sem=y_recv.at[s],
                device_id=(my_x, my_y ^ dist, my_z),
                device_id_type=pl.DeviceIdType.MESH,
            )
            rdma.start()
            rdma.wait()
            slab_ref[:, :] = slab_ref[:, :] + ycomm_ref[s, :, :]

        z_rdmas = []
        for d in (1, 2, 3):
            r = pltpu.make_async_remote_copy(
                src_ref=slab_ref,
                dst_ref=out_ref.at[pl.ds(row0, MS), :],
                send_sem=z_send.at[d - 1],
                recv_sem=z_recv.at[d - 1],
                device_id=(my_x, my_y, (my_z + d) % NZ),
                device_id_type=pl.DeviceIdType.MESH,
            )
            r.start()
            z_rdmas.append(r)
        x_own = pltpu.make_async_remote_copy(
            src_ref=slab_ref,
            dst_ref=out_ref.at[pl.ds(row0, MS), :],
            send_sem=x_send.at[0],
            recv_sem=x_recv.at[0],
            device_id=(1 - my_x, my_y, my_z),
            device_id_type=pl.DeviceIdType.MESH,
        )
        x_own.start()
        out_ref[pl.ds(row0, MS), :] = slab_ref[:, :]

        x_fwds = []
        for d in (1, 2, 3):
            orig = MC * ((my_z - d) % NZ) + MS * my_x
            wait = pltpu.make_async_remote_copy(
                src_ref=slab_ref,
                dst_ref=out_ref.at[pl.ds(orig, MS), :],
                send_sem=z_send.at[d - 1],
                recv_sem=z_recv.at[d - 1],
                device_id=(my_x, my_y, my_z),
                device_id_type=pl.DeviceIdType.MESH,
            )
            wait.wait_recv()
            f = pltpu.make_async_remote_copy(
                src_ref=out_ref.at[pl.ds(orig, MS), :],
                dst_ref=out_ref.at[pl.ds(orig, MS), :],
                send_sem=x_send.at[d],
                recv_sem=x_recv.at[d],
                device_id=(1 - my_x, my_y, my_z),
                device_id_type=pl.DeviceIdType.MESH,
            )
            f.start()
            x_fwds.append(f)

        for d in (0, 1, 2, 3):
            orig = MC * ((my_z - d) % NZ) + MS * (1 - my_x)
            wait = pltpu.make_async_remote_copy(
                src_ref=slab_ref,
                dst_ref=out_ref.at[pl.ds(orig, MS), :],
                send_sem=x_send.at[d],
                recv_sem=x_recv.at[d],
                device_id=(1 - my_x, my_y, my_z),
                device_id_type=pl.DeviceIdType.MESH,
            )
            wait.wait_recv()

        for r in z_rdmas:
            r.wait_send()
        x_own.wait_send()
        for f in x_fwds:
            f.wait_send()

    return pl.pallas_call(
        body,
        out_shape=jax.ShapeDtypeStruct((M, D), jnp.float32),
        in_specs=[
            pl.BlockSpec(memory_space=pl.ANY),
            pl.BlockSpec(memory_space=pltpu.VMEM),
        ],
        out_specs=pl.BlockSpec(memory_space=pltpu.VMEM),
        scratch_shapes=[
            pltpu.VMEM((MS, K), jnp.float32),
            pltpu.VMEM((MS, D), jnp.float32),
            pltpu.VMEM((2, MS, D), jnp.float32),
            pltpu.SemaphoreType.DMA,
            pltpu.SemaphoreType.DMA((2,)),
            pltpu.SemaphoreType.DMA((2,)),
            pltpu.SemaphoreType.DMA((3,)),
            pltpu.SemaphoreType.DMA((3,)),
            pltpu.SemaphoreType.DMA((4,)),
            pltpu.SemaphoreType.DMA((4,)),
        ],
        compiler_params=pltpu.CompilerParams(collective_id=0),
    )(dy, W)


# device time: 26301 ns/iter; 1.1296x vs baseline; 1.1296x over previous
import jax
import jax.numpy as jnp
from jax import lax
from jax.experimental import pallas as pl
from jax.experimental.pallas import tpu as pltpu

M = 512
D = 512
K = 2048
MS = 64
MC = 128
CH = 256
NY = 4
NZ = 4


def kernel(dy, W):
    def body(
        dy_ref, w_ref, out_ref,
        dyslab_ref, slab_ref, ycomm_ref,
        copy_sem, y_send, y_recv,
        z_send, z_recv, xo_send, xo_recv, xf_send, xf_recv,
    ):
        my_x = lax.axis_index("x")
        my_y = lax.axis_index("y")
        my_z = lax.axis_index("z")
        row0 = MC * my_z + MS * my_x

        dy_copy = pltpu.make_async_copy(
            dy_ref.at[pl.ds(row0, MS), :], dyslab_ref, copy_sem
        )
        dy_copy.start()

        barrier_sem = pltpu.get_barrier_semaphore()
        for d in (1, 2, 3):
            pl.semaphore_signal(
                barrier_sem, inc=1,
                device_id=(my_x, (my_y + d) % NY, my_z),
                device_id_type=pl.DeviceIdType.MESH,
            )
        pl.semaphore_signal(
            barrier_sem, inc=1,
            device_id=(1 - my_x, my_y, my_z),
            device_id_type=pl.DeviceIdType.MESH,
        )
        for d in (1, 2, 3):
            pl.semaphore_signal(
                barrier_sem, inc=1,
                device_id=(my_x, my_y, (my_z + d) % NZ),
                device_id_type=pl.DeviceIdType.MESH,
            )

        dy_copy.wait()

        slab_ref[:, :] = lax.dot_general(
            dyslab_ref[:, :], w_ref[:, :],
            (((1,), (1,)), ((), ())),
            preferred_element_type=jnp.float32,
        )

        pl.semaphore_wait(barrier_sem, 7)

        y_rdmas = []
        for h in (0, 1):
            for d in (1, 2, 3):
                r = pltpu.make_async_remote_copy(
                    src_ref=slab_ref.at[:, pl.ds(h * CH, CH)],
                    dst_ref=ycomm_ref.at[h * 3 + d - 1],
                    send_sem=y_send.at[h * 3 + d - 1],
                    recv_sem=y_recv.at[h * 3 + d - 1],
                    device_id=(my_x, (my_y + d) % NY, my_z),
                    device_id_type=pl.DeviceIdType.MESH,
                )
                r.start()
                y_rdmas.append(r)

        z_rdmas = []
        for h in (0, 1):
            for d in (1, 2, 3):
                y_rdmas[h * 3 + d - 1].wait_recv()
            out_ref[pl.ds(row0, MS), pl.ds(h * CH, CH)] = (
                (slab_ref[:, pl.ds(h * CH, CH)] + ycomm_ref[h * 3 + 0, :, :])
                + (ycomm_ref[h * 3 + 1, :, :] + ycomm_ref[h * 3 + 2, :, :])
            )
            for d in (1, 2, 3):
                r = pltpu.make_async_remote_copy(
                    src_ref=out_ref.at[pl.ds(row0, MS), pl.ds(h * CH, CH)],
                    dst_ref=out_ref.at[pl.ds(row0, MS), pl.ds(h * CH, CH)],
                    send_sem=z_send.at[h * 3 + d - 1],
                    recv_sem=z_recv.at[h * 3 + d - 1],
                    device_id=(my_x, my_y, (my_z + d) % NZ),
                    device_id_type=pl.DeviceIdType.MESH,
                )
                r.start()
                z_rdmas.append(r)

        x_own = pltpu.make_async_remote_copy(
            src_ref=out_ref.at[pl.ds(row0, MS), :],
            dst_ref=out_ref.at[pl.ds(row0, MS), :],
            send_sem=xo_send,
            recv_sem=xo_recv,
            device_id=(1 - my_x, my_y, my_z),
            device_id_type=pl.DeviceIdType.MESH,
        )
        x_own.start()

        x_fwds = []
        for h in (0, 1):
            for d in (1, 2, 3):
                orig = MC * ((my_z - d) % NZ) + MS * my_x
                wait = pltpu.make_async_remote_copy(
                    src_ref=slab_ref.at[:, pl.ds(h * CH, CH)],
                    dst_ref=out_ref.at[pl.ds(orig, MS), pl.ds(h * CH, CH)],
                    send_sem=z_send.at[h * 3 + d - 1],
                    recv_sem=z_recv.at[h * 3 + d - 1],
                    device_id=(my_x, my_y, my_z),
                    device_id_type=pl.DeviceIdType.MESH,
                )
                wait.wait_recv()
                f = pltpu.make_async_remote_copy(
                    src_ref=out_ref.at[pl.ds(orig, MS), pl.ds(h * CH, CH)],
                    dst_ref=out_ref.at[pl.ds(orig, MS), pl.ds(h * CH, CH)],
                    send_sem=xf_send.at[h * 3 + d - 1],
                    recv_sem=xf_recv.at[h * 3 + d - 1],
                    device_id=(1 - my_x, my_y, my_z),
                    device_id_type=pl.DeviceIdType.MESH,
                )
                f.start()
                x_fwds.append(f)

        xo_wait = pltpu.make_async_remote_copy(
            src_ref=out_ref.at[pl.ds(MC * my_z + MS * (1 - my_x), MS), :],
            dst_ref=out_ref.at[pl.ds(MC * my_z + MS * (1 - my_x), MS), :],
            send_sem=xo_send,
            recv_sem=xo_recv,
            device_id=(1 - my_x, my_y, my_z),
            device_id_type=pl.DeviceIdType.MESH,
        )
        xo_wait.wait_recv()
        for h in (0, 1):
            for d in (1, 2, 3):
                orig = MC * ((my_z - d) % NZ) + MS * (1 - my_x)
                wait = pltpu.make_async_remote_copy(
                    src_ref=slab_ref.at[:, pl.ds(h * CH, CH)],
                    dst_ref=out_ref.at[pl.ds(orig, MS), pl.ds(h * CH, CH)],
                    send_sem=xf_send.at[h * 3 + d - 1],
                    recv_sem=xf_recv.at[h * 3 + d - 1],
                    device_id=(1 - my_x, my_y, my_z),
                    device_id_type=pl.DeviceIdType.MESH,
                )
                wait.wait_recv()

        for r in y_rdmas:
            r.wait_send()
        for r in z_rdmas:
            r.wait_send()
        x_own.wait_send()
        for f in x_fwds:
            f.wait_send()

    return pl.pallas_call(
        body,
        out_shape=jax.ShapeDtypeStruct((M, D), jnp.float32),
        in_specs=[
            pl.BlockSpec(memory_space=pl.ANY),
            pl.BlockSpec(memory_space=pltpu.VMEM),
        ],
        out_specs=pl.BlockSpec(memory_space=pltpu.VMEM),
        scratch_shapes=[
            pltpu.VMEM((MS, K), jnp.float32),
            pltpu.VMEM((MS, D), jnp.float32),
            pltpu.VMEM((6, MS, CH), jnp.float32),
            pltpu.SemaphoreType.DMA,
            pltpu.SemaphoreType.DMA((6,)),
            pltpu.SemaphoreType.DMA((6,)),
            pltpu.SemaphoreType.DMA((6,)),
            pltpu.SemaphoreType.DMA((6,)),
            pltpu.SemaphoreType.DMA,
            pltpu.SemaphoreType.DMA,
            pltpu.SemaphoreType.DMA((6,)),
            pltpu.SemaphoreType.DMA((6,)),
        ],
        compiler_params=pltpu.CompilerParams(collective_id=0),
    )(dy, W)
